# Initial kernel scaffold; baseline (speedup 1.0000x reference)
import jax
import jax.numpy as jnp
from jax.experimental import pallas as pl


def kernel(x, encoding_weight):
    raise NotImplementedError("write your pallas kernel here")


# speedup vs baseline: 1.9246x; 1.9246x over previous
import jax
import jax.numpy as jnp
from jax import lax
from jax.experimental import pallas as pl
from jax.experimental.pallas import tpu as pltpu
from jax.experimental.pallas import tpu_sc as plsc

_BATCH = 16384
_TD = 20 * 128
_NC, _NS = 2, 16
_NW = _NC * _NS
_RW = _BATCH // _NW
_C = 16
_S = _RW // _C
_L = 16
_VR = _TD // _L


def _body(x_hbm, w_hbm, out_hbm, w_v, buf):
    wid = lax.axis_index("s") * _NC + lax.axis_index("c")
    base = wid * _RW
    pltpu.sync_copy(w_hbm, w_v)

    def step(s, carry):
        row0 = base + s * _C
        pltpu.sync_copy(x_hbm.at[pl.ds(row0, _C)], buf)

        def jstep(j, c2):
            wv = w_v[pl.ds(j * _L, _L)]
            for r in range(_C):
                plsc.addupdate(buf.at[r, pl.ds(j * _L, _L)], wv)
            return c2

        lax.fori_loop(0, _VR, jstep, 0, unroll=2)
        pltpu.sync_copy(buf, out_hbm.at[pl.ds(row0, _C)])
        return carry

    lax.fori_loop(0, _S, step, 0)


@jax.jit
def _role_add(x2, w2):
    mesh = plsc.VectorSubcoreMesh(
        core_axis_name="c", subcore_axis_name="s",
        num_cores=_NC, num_subcores=_NS)
    return pl.kernel(
        _body,
        out_type=jax.ShapeDtypeStruct((_BATCH, _TD), jnp.float32),
        mesh=mesh,
        scratch_types=[
            pltpu.VMEM((_TD,), jnp.float32),
            pltpu.VMEM((_C, _TD), jnp.float32),
        ],
    )(x2, w2)


def kernel(x, encoding_weight):
    b, t, d = x.shape
    x2 = x.reshape(b, t * d)
    w2 = encoding_weight.reshape(t * d)
    return _role_add(x2, w2).reshape(b, t, d)


# speedup vs baseline: 1.9473x; 1.0118x over previous
import jax
import jax.numpy as jnp
from jax import lax
from jax.experimental import pallas as pl
from jax.experimental.pallas import tpu as pltpu
from jax.experimental.pallas import tpu_sc as plsc

_BATCH = 16384
_TD = 20 * 128
_NC, _NS = 2, 16
_NW = _NC * _NS
_RW = _BATCH // _NW
_C = 8
_S = _RW // _C
_L = 16
_VR = _TD // _L


def _body(x_hbm, w_hbm, out_hbm, w_v, in0, in1, ou0, ou1, si0, si1, so0, so1):
    ins, outs = (in0, in1), (ou0, ou1)
    sis, sos = (si0, si1), (so0, so1)
    wid = lax.axis_index("s") * _NC + lax.axis_index("c")
    base = wid * _RW
    pltpu.sync_copy(w_hbm, w_v)

    def start_in(s, b):
        pltpu.async_copy(x_hbm.at[pl.ds(base + s * _C, _C)], ins[b], sis[b])

    def wait_in(b):
        pltpu.make_async_copy(x_hbm.at[pl.ds(base, _C)], ins[b], sis[b]).wait()

    def start_out(s, b):
        pltpu.async_copy(outs[b], out_hbm.at[pl.ds(base + s * _C, _C)], sos[b])

    def wait_out(b):
        pltpu.make_async_copy(outs[b], out_hbm.at[pl.ds(base, _C)], sos[b]).wait()

    def compute(b):
        def jstep(j, c):
            wv = w_v[pl.ds(j * _L, _L)]
            for r in range(_C):
                outs[b][r, pl.ds(j * _L, _L)] = ins[b][r, pl.ds(j * _L, _L)] + wv
            return c
        lax.fori_loop(0, _VR, jstep, 0, unroll=2)

    start_in(0, 0)
    start_in(1, 1)
    for b in range(2):
        wait_in(b)
        compute(b)
        start_out(b, b)
        start_in(b + 2, b)

    def gstep(g, c):
        for b in range(2):
            s = g * 2 + b
            wait_out(b)
            wait_in(b)
            compute(b)
            start_out(s, b)
            start_in(s + 2, b)
        return c

    lax.fori_loop(1, _S // 2 - 1, gstep, 0)

    for b in range(2):
        s = _S - 2 + b
        wait_out(b)
        wait_in(b)
        compute(b)
        start_out(s, b)
    wait_out(0)
    wait_out(1)


@jax.jit
def _role_add(x2, w2):
    mesh = plsc.VectorSubcoreMesh(
        core_axis_name="c", subcore_axis_name="s",
        num_cores=_NC, num_subcores=_NS)
    return pl.kernel(
        _body,
        out_type=jax.ShapeDtypeStruct((_BATCH, _TD), jnp.float32),
        mesh=mesh,
        scratch_types=[
            pltpu.VMEM((_TD,), jnp.float32),
            pltpu.VMEM((_C, _TD), jnp.float32),
            pltpu.VMEM((_C, _TD), jnp.float32),
            pltpu.VMEM((_C, _TD), jnp.float32),
            pltpu.VMEM((_C, _TD), jnp.float32),
            pltpu.SemaphoreType.DMA,
            pltpu.SemaphoreType.DMA,
            pltpu.SemaphoreType.DMA,
            pltpu.SemaphoreType.DMA,
        ],
    )(x2, w2)


def kernel(x, encoding_weight):
    b, t, d = x.shape
    x2 = x.reshape(b, t * d)
    w2 = encoding_weight.reshape(t * d)
    return _role_add(x2, w2).reshape(b, t, d)


# speedup vs baseline: 3.1946x; 1.6405x over previous
import jax
import jax.numpy as jnp
from jax import lax
from jax.experimental import pallas as pl
from jax.experimental.pallas import tpu as pltpu
from jax.experimental.pallas import tpu_sc as plsc

_BATCH = 16384
_T, _D = 20, 128
_NC, _NS = 2, 16
_NW = _NC * _NS
_RW = _BATCH // _NW
_C = 8
_S = _RW // _C
_L = 16
_KD = _D // _L


def _body(x_hbm, w_hbm, out_hbm, w_v, in0, in1, ou0, ou1, si0, si1, so0, so1):
    ins, outs = (in0, in1), (ou0, ou1)
    sis, sos = (si0, si1), (so0, so1)
    wid = lax.axis_index("s") * _NC + lax.axis_index("c")
    base = wid * _RW
    pltpu.sync_copy(w_hbm, w_v)

    def start_in(s, b):
        pltpu.async_copy(x_hbm.at[pl.ds(base + s * _C, _C)], ins[b], sis[b])

    def wait_in(b):
        pltpu.make_async_copy(x_hbm.at[pl.ds(base, _C)], ins[b], sis[b]).wait()

    def start_out(s, b):
        pltpu.async_copy(outs[b], out_hbm.at[pl.ds(base + s * _C, _C)], sos[b])

    def wait_out(b):
        pltpu.make_async_copy(outs[b], out_hbm.at[pl.ds(base, _C)], sos[b]).wait()

    def compute(b):
        def tstep(t, c):
            for k in range(_KD):
                wv = w_v[t, pl.ds(k * _L, _L)]
                for r in range(_C):
                    outs[b][r, t, pl.ds(k * _L, _L)] = (
                        ins[b][r, t, pl.ds(k * _L, _L)] + wv)
            return c
        lax.fori_loop(0, _T, tstep, 0)

    start_in(0, 0)
    start_in(1, 1)
    for b in range(2):
        wait_in(b)
        compute(b)
        start_out(b, b)
        start_in(b + 2, b)

    def gstep(g, c):
        for b in range(2):
            s = g * 2 + b
            wait_out(b)
            wait_in(b)
            compute(b)
            start_out(s, b)
            start_in(s + 2, b)
        return c

    lax.fori_loop(1, _S // 2 - 1, gstep, 0)

    for b in range(2):
        s = _S - 2 + b
        wait_out(b)
        wait_in(b)
        compute(b)
        start_out(s, b)
    wait_out(0)
    wait_out(1)


@jax.jit
def _role_add(x, w):
    mesh = plsc.VectorSubcoreMesh(
        core_axis_name="c", subcore_axis_name="s",
        num_cores=_NC, num_subcores=_NS)
    return pl.kernel(
        _body,
        out_type=jax.ShapeDtypeStruct((_BATCH, _T, _D), jnp.float32),
        mesh=mesh,
        compiler_params=pltpu.CompilerParams(use_tc_tiling_on_sc=True),
        scratch_types=[
            pltpu.VMEM((_T, _D), jnp.float32),
            pltpu.VMEM((_C, _T, _D), jnp.float32),
            pltpu.VMEM((_C, _T, _D), jnp.float32),
            pltpu.VMEM((_C, _T, _D), jnp.float32),
            pltpu.VMEM((_C, _T, _D), jnp.float32),
            pltpu.SemaphoreType.DMA,
            pltpu.SemaphoreType.DMA,
            pltpu.SemaphoreType.DMA,
            pltpu.SemaphoreType.DMA,
        ],
    )(x, w)


def kernel(x, encoding_weight):
    return _role_add(x, encoding_weight)
